# baseline (speedup 1.0000x reference)
import jax
import jax.numpy as jnp
from jax.experimental import pallas as pl
from jax.experimental.pallas import tpu as pltpu

EPS = 1e-5
N_HEADS = 8


def _ln(x, g, b):
    m = jnp.mean(x, axis=-1, keepdims=True)
    xc = x - m
    v = jnp.mean(xc * xc, axis=-1, keepdims=True)
    return xc * jax.lax.rsqrt(v + EPS) * g + b


def _mm(a, w):
    return jnp.dot(a, w, preferred_element_type=jnp.float32)


def _mha(qb_in, kvb_in, wq, bq, wk, bk, wv, bv, wo, bo, attn_ref):
    D = qb_in.shape[-1]
    dh = D // N_HEADS
    scale = 1.0 / jnp.sqrt(jnp.float32(dh))
    bf16 = jnp.bfloat16
    q = (_mm(qb_in, wq) + bq).astype(bf16)
    k = (_mm(kvb_in, wk) + bk).astype(bf16)
    v = (_mm(kvb_in, wv) + bv).astype(bf16)
    outs = []
    for h in range(N_HEADS):
        qh = q[:, h * dh:(h + 1) * dh]
        kh = k[:, h * dh:(h + 1) * dh]
        vh = v[:, h * dh:(h + 1) * dh]
        s = jax.lax.dot_general(qh, kh, (((1,), (1,)), ((), ())),
                                preferred_element_type=jnp.float32) * scale
        m = jnp.max(s, axis=1, keepdims=True)
        e = jnp.exp(s - m)
        denom = jnp.sum(e, axis=1, keepdims=True)
        p = e / denom
        if attn_ref is not None:
            attn_ref[0, h] = p
        outs.append(_mm(p.astype(bf16), vh))
    o = jnp.concatenate(outs, axis=1).astype(bf16)
    return _mm(o, wo) + bo


def _make_layer_body(first, last):
    def body(*refs):
        it = iter(refs)
        if first:
            trg_ref = next(it)
            ftw_ref = next(it)
            ftb_ref = next(it)
        else:
            x_ref = next(it)
        src_ref = next(it)
        sa = [next(it) for _ in range(8)]
        ea = [next(it) for _ in range(8)]
        pfw1, pfb1, pfw2, pfb2 = next(it), next(it), next(it), next(it)
        ln1g, ln1b, ln2g, ln2b, ln3g, ln3b = (next(it) for _ in range(6))
        if last:
            fc1w, fc1b, fc2w, fc2b, fc3w, fc3b = (next(it) for _ in range(6))
            attn_ref = next(it)
            pooled_ref = next(it)
            label_ref = next(it)
        else:
            xo_ref = next(it)

        bf16 = jnp.bfloat16
        if first:
            x = _mm(trg_ref[0], ftw_ref[...]) + ftb_ref[...]
        else:
            x = x_ref[0]

        xb = x.astype(bf16)
        s_out = _mha(xb, xb, sa[0][0], sa[1][0], sa[2][0], sa[3][0],
                     sa[4][0], sa[5][0], sa[6][0], sa[7][0], None)
        x = _ln(x + s_out, ln1g[0], ln1b[0])
        c_out = _mha(x.astype(bf16), src_ref[0], ea[0][0], ea[1][0],
                     ea[2][0], ea[3][0], ea[4][0], ea[5][0], ea[6][0],
                     ea[7][0], attn_ref if last else None)
        x = _ln(x + c_out, ln2g[0], ln2b[0])
        f = jnp.maximum(_mm(x.astype(bf16), pfw1[0]) + pfb1[0], 0.0)
        f = _mm(f.astype(bf16), pfw2[0]) + pfb2[0]
        x = _ln(x + f, ln3g[0], ln3b[0])

        if last:
            sq = jnp.sum(x * x, axis=1, keepdims=True)
            nrm = jnp.sqrt(sq)
            mx = jnp.max(nrm, axis=0, keepdims=True)
            e = jnp.exp(nrm - mx)
            w = e / jnp.sum(e, axis=0, keepdims=True)
            pooled = jax.lax.dot_general(w, x, (((0,), (0,)), ((), ())),
                                         preferred_element_type=jnp.float32)
            h1 = jnp.maximum(
                jnp.dot(pooled, fc1w[...],
                        preferred_element_type=jnp.float32) + fc1b[...], 0.0)
            h2 = jnp.maximum(
                jnp.dot(h1, fc2w[...],
                        preferred_element_type=jnp.float32) + fc2b[...], 0.0)
            lab = (jnp.dot(h2, fc3w[...],
                           preferred_element_type=jnp.float32) + fc3b[...])
            pooled_ref[0] = pooled
            label_ref[0] = lab
        else:
            xo_ref[0] = x

    return body


def kernel(trg, src, ft_w, ft_b,
           sa_wq, sa_bq, sa_wk, sa_bk, sa_wv, sa_bv, sa_wo, sa_bo,
           ea_wq, ea_bq, ea_wk, ea_bk, ea_wv, ea_bv, ea_wo, ea_bo,
           pf_w1, pf_b1, pf_w2, pf_b2,
           ln1_g, ln1_b, ln2_g, ln2_b, ln3_g, ln3_b,
           fc1_w, fc1_b, fc2_w, fc2_b, fc3_w, fc3_b):
    B, St, LOCAL = trg.shape
    Ss, D = src.shape[1], src.shape[2]
    L = sa_wq.shape[0]
    F = pf_w1.shape[2]
    f32 = jnp.float32

    bf16 = jnp.bfloat16
    trg = trg.astype(bf16)
    src = src.astype(bf16)
    ft_w = ft_w.astype(bf16)
    sa_wq, sa_wk, sa_wv, sa_wo = (a.astype(bf16)
                                  for a in (sa_wq, sa_wk, sa_wv, sa_wo))
    ea_wq, ea_wk, ea_wv, ea_wo = (a.astype(bf16)
                                  for a in (ea_wq, ea_wk, ea_wv, ea_wo))
    pf_w1, pf_w2 = pf_w1.astype(bf16), pf_w2.astype(bf16)

    b3 = lambda a: a.reshape(L, 1, a.shape[-1])
    sa_bq3, sa_bk3, sa_bv3, sa_bo3 = map(b3, (sa_bq, sa_bk, sa_bv, sa_bo))
    ea_bq3, ea_bk3, ea_bv3, ea_bo3 = map(b3, (ea_bq, ea_bk, ea_bv, ea_bo))
    pf_b13, pf_b23 = b3(pf_b1), b3(pf_b2)
    ln1_g3, ln1_b3, ln2_g3, ln2_b3, ln3_g3, ln3_b3 = map(
        b3, (ln1_g, ln1_b, ln2_g, ln2_b, ln3_g, ln3_b))
    ft_b2 = ft_b.reshape(1, D)
    fc1_b2, fc2_b2, fc3_b2 = (fc1_b.reshape(1, -1), fc2_b.reshape(1, -1),
                              fc3_b.reshape(1, -1))

    BH = B // 2

    def bidx(c, i):
        return c * BH + i

    def wspec(l, shape):
        n = len(shape)
        return pl.BlockSpec((1,) + shape, lambda c, i, l=l: (l,) + (0,) * n)

    def full(shape):
        n = len(shape)
        return pl.BlockSpec(shape, lambda c, i: (0,) * n)

    x = None
    for l in range(L):
        first, last = l == 0, l == L - 1
        ins = []
        in_specs = []
        if first:
            ins += [trg, ft_w, ft_b2]
            in_specs += [pl.BlockSpec((1, St, LOCAL), lambda c, i: (bidx(c, i), 0, 0)),
                         full((LOCAL, D)), full((1, D))]
        else:
            ins += [x]
            in_specs += [pl.BlockSpec((1, St, D), lambda c, i: (bidx(c, i), 0, 0))]
        ins += [src]
        in_specs += [pl.BlockSpec((1, Ss, D), lambda c, i: (bidx(c, i), 0, 0))]
        for w_, b_ in ((sa_wq, sa_bq3), (sa_wk, sa_bk3), (sa_wv, sa_bv3),
                       (sa_wo, sa_bo3)):
            ins += [w_, b_]
            in_specs += [wspec(l, (D, D)), wspec(l, (1, D))]
        for w_, b_ in ((ea_wq, ea_bq3), (ea_wk, ea_bk3), (ea_wv, ea_bv3),
                       (ea_wo, ea_bo3)):
            ins += [w_, b_]
            in_specs += [wspec(l, (D, D)), wspec(l, (1, D))]
        ins += [pf_w1, pf_b13, pf_w2, pf_b23]
        in_specs += [wspec(l, (D, F)), wspec(l, (1, F)),
                     wspec(l, (F, D)), wspec(l, (1, D))]
        for p_ in (ln1_g3, ln1_b3, ln2_g3, ln2_b3, ln3_g3, ln3_b3):
            ins += [p_]
            in_specs += [wspec(l, (1, D))]
        if last:
            ins += [fc1_w, fc1_b2, fc2_w, fc2_b2, fc3_w, fc3_b2]
            in_specs += [full(fc1_w.shape), full((1, fc1_w.shape[1])),
                         full(fc2_w.shape), full((1, fc2_w.shape[1])),
                         full(fc3_w.shape), full((1, fc3_w.shape[1]))]
            out_shape = [jax.ShapeDtypeStruct((B, N_HEADS, St, Ss), f32),
                         jax.ShapeDtypeStruct((B, 1, D), f32),
                         jax.ShapeDtypeStruct((B, 1, 2), f32)]
            out_specs = [pl.BlockSpec((1, N_HEADS, St, Ss),
                                      lambda c, i: (bidx(c, i), 0, 0, 0)),
                         pl.BlockSpec((1, 1, D), lambda c, i: (bidx(c, i), 0, 0)),
                         pl.BlockSpec((1, 1, 2), lambda c, i: (bidx(c, i), 0, 0))]
        else:
            out_shape = jax.ShapeDtypeStruct((B, St, D), f32)
            out_specs = pl.BlockSpec((1, St, D), lambda c, i: (bidx(c, i), 0, 0))

        res = pl.pallas_call(
            _make_layer_body(first, last),
            grid=(2, BH),
            in_specs=in_specs,
            out_specs=out_specs,
            out_shape=out_shape,
            compiler_params=pltpu.CompilerParams(
                dimension_semantics=("parallel", "arbitrary"),
                vmem_limit_bytes=56 * 1024 * 1024,
            ),
        )(*ins)
        if last:
            attn, pooled3, label3 = res
        else:
            x = res

    return pooled3.reshape(B, D), attn, label3.reshape(B, 2)


# speedup vs baseline: 1.0807x; 1.0807x over previous
import jax
import jax.numpy as jnp
from jax.experimental import pallas as pl
from jax.experimental.pallas import tpu as pltpu

EPS = 1e-5
N_HEADS = 8


def _ln(x, g, b):
    m = jnp.mean(x, axis=-1, keepdims=True)
    msq = jnp.mean(x * x, axis=-1, keepdims=True)
    v = msq - m * m
    return (x - m) * jax.lax.rsqrt(v + EPS) * g + b


def _mm(a, w):
    return jnp.dot(a, w, preferred_element_type=jnp.float32)


def _mha(q_in, kv_in, G, wq, bq, wk, bk, wv, bv, wo, bo, attn_ref):
    D = q_in.shape[-1]
    Sq = q_in.shape[0] // G
    Sk = kv_in.shape[0] // G
    dh = D // N_HEADS
    scale = 1.0 / jnp.sqrt(jnp.float32(dh))
    q = _mm(q_in, wq) + bq
    k = _mm(kv_in, wk) + bk
    v = _mm(kv_in, wv) + bv
    rows = []
    for g in range(G):
        outs = []
        for h in range(N_HEADS):
            qh = q[g * Sq:(g + 1) * Sq, h * dh:(h + 1) * dh]
            kh = k[g * Sk:(g + 1) * Sk, h * dh:(h + 1) * dh]
            vh = v[g * Sk:(g + 1) * Sk, h * dh:(h + 1) * dh]
            s = jax.lax.dot_general(qh, kh, (((1,), (1,)), ((), ())),
                                    preferred_element_type=jnp.float32)
            s = s * scale
            m = jnp.max(s, axis=1, keepdims=True)
            e = jnp.exp(s - m)
            denom = jnp.sum(e, axis=1, keepdims=True)
            r = 1.0 / denom
            if attn_ref is not None:
                attn_ref[g, h] = e * r
            outs.append(_mm(e, vh) * r)
        rows.append(jnp.concatenate(outs, axis=1))
    o = jnp.concatenate(rows, axis=0) if G > 1 else rows[0]
    return _mm(o, wo) + bo


def _make_layer_body(first, last, G):
    def body(*refs):
        it = iter(refs)
        if first:
            trg_ref = next(it)
            ftw_ref = next(it)
            ftb_ref = next(it)
        else:
            x_ref = next(it)
        src_ref = next(it)
        sa = [next(it) for _ in range(8)]
        ea = [next(it) for _ in range(8)]
        pfw1, pfb1, pfw2, pfb2 = next(it), next(it), next(it), next(it)
        ln1g, ln1b, ln2g, ln2b, ln3g, ln3b = (next(it) for _ in range(6))
        if last:
            fc1w, fc1b, fc2w, fc2b, fc3w, fc3b = (next(it) for _ in range(6))
            attn_ref = next(it)
            pooled_ref = next(it)
            label_ref = next(it)
        else:
            xo_ref = next(it)

        if first:
            t = trg_ref[...]
            t = t.reshape(t.shape[0] * t.shape[1], t.shape[2])
            x = _mm(t, ftw_ref[...]) + ftb_ref[...]
        else:
            xi = x_ref[...]
            x = xi.reshape(xi.shape[0] * xi.shape[1], xi.shape[2])
        sc = src_ref[...]
        src2 = sc.reshape(sc.shape[0] * sc.shape[1], sc.shape[2])

        s_out = _mha(x, x, G, sa[0][0], sa[1][0], sa[2][0], sa[3][0],
                     sa[4][0], sa[5][0], sa[6][0], sa[7][0], None)
        x = _ln(x + s_out, ln1g[0], ln1b[0])
        c_out = _mha(x, src2, G, ea[0][0], ea[1][0], ea[2][0], ea[3][0],
                     ea[4][0], ea[5][0], ea[6][0], ea[7][0],
                     attn_ref if last else None)
        x = _ln(x + c_out, ln2g[0], ln2b[0])
        f = jnp.maximum(_mm(x, pfw1[0]) + pfb1[0], 0.0)
        f = _mm(f, pfw2[0]) + pfb2[0]
        x = _ln(x + f, ln3g[0], ln3b[0])

        if last:
            sq = jnp.sum(x * x, axis=1, keepdims=True)
            nrm = jnp.sqrt(sq)
            mx = jnp.max(nrm, axis=0, keepdims=True)
            e = jnp.exp(nrm - mx)
            w = e / jnp.sum(e, axis=0, keepdims=True)
            pooled = jax.lax.dot_general(w, x, (((0,), (0,)), ((), ())),
                                         preferred_element_type=jnp.float32)
            h1 = jnp.maximum(_mm(pooled, fc1w[...]) + fc1b[...], 0.0)
            h2 = jnp.maximum(_mm(h1, fc2w[...]) + fc2b[...], 0.0)
            lab = _mm(h2, fc3w[...]) + fc3b[...]
            pooled_ref[0] = pooled
            label_ref[0] = lab
        else:
            St = xo_ref.shape[1]
            xo_ref[...] = x.reshape(G, St, x.shape[1])

    return body


def kernel(trg, src, ft_w, ft_b,
           sa_wq, sa_bq, sa_wk, sa_bk, sa_wv, sa_bv, sa_wo, sa_bo,
           ea_wq, ea_bq, ea_wk, ea_bk, ea_wv, ea_bv, ea_wo, ea_bo,
           pf_w1, pf_b1, pf_w2, pf_b2,
           ln1_g, ln1_b, ln2_g, ln2_b, ln3_g, ln3_b,
           fc1_w, fc1_b, fc2_w, fc2_b, fc3_w, fc3_b):
    B, St, LOCAL = trg.shape
    Ss, D = src.shape[1], src.shape[2]
    L = sa_wq.shape[0]
    F = pf_w1.shape[2]
    f32 = jnp.float32

    b3 = lambda a: a.reshape(L, 1, a.shape[-1])
    sa_bq3, sa_bk3, sa_bv3, sa_bo3 = map(b3, (sa_bq, sa_bk, sa_bv, sa_bo))
    ea_bq3, ea_bk3, ea_bv3, ea_bo3 = map(b3, (ea_bq, ea_bk, ea_bv, ea_bo))
    pf_b13, pf_b23 = b3(pf_b1), b3(pf_b2)
    ln1_g3, ln1_b3, ln2_g3, ln2_b3, ln3_g3, ln3_b3 = map(
        b3, (ln1_g, ln1_b, ln2_g, ln2_b, ln3_g, ln3_b))
    ft_b2 = ft_b.reshape(1, D)
    fc1_b2, fc2_b2, fc3_b2 = (fc1_b.reshape(1, -1), fc2_b.reshape(1, -1),
                              fc3_b.reshape(1, -1))

    def wspec(l, shape):
        n = len(shape)
        return pl.BlockSpec((1,) + shape, lambda b, l=l: (l,) + (0,) * n)

    def full(shape):
        n = len(shape)
        return pl.BlockSpec(shape, lambda b: (0,) * n)

    x = None
    for l in range(L):
        first, last = l == 0, l == L - 1
        G = 1 if last else 2
        ins = []
        in_specs = []
        if first:
            ins += [trg, ft_w, ft_b2]
            in_specs += [pl.BlockSpec((G, St, LOCAL), lambda b: (b, 0, 0)),
                         full((LOCAL, D)), full((1, D))]
        else:
            ins += [x]
            in_specs += [pl.BlockSpec((G, St, D), lambda b: (b, 0, 0))]
        ins += [src]
        in_specs += [pl.BlockSpec((G, Ss, D), lambda b: (b, 0, 0))]
        for w_, b_ in ((sa_wq, sa_bq3), (sa_wk, sa_bk3), (sa_wv, sa_bv3),
                       (sa_wo, sa_bo3)):
            ins += [w_, b_]
            in_specs += [wspec(l, (D, D)), wspec(l, (1, D))]
        for w_, b_ in ((ea_wq, ea_bq3), (ea_wk, ea_bk3), (ea_wv, ea_bv3),
                       (ea_wo, ea_bo3)):
            ins += [w_, b_]
            in_specs += [wspec(l, (D, D)), wspec(l, (1, D))]
        ins += [pf_w1, pf_b13, pf_w2, pf_b23]
        in_specs += [wspec(l, (D, F)), wspec(l, (1, F)),
                     wspec(l, (F, D)), wspec(l, (1, D))]
        for p_ in (ln1_g3, ln1_b3, ln2_g3, ln2_b3, ln3_g3, ln3_b3):
            ins += [p_]
            in_specs += [wspec(l, (1, D))]
        if last:
            ins += [fc1_w, fc1_b2, fc2_w, fc2_b2, fc3_w, fc3_b2]
            in_specs += [full(fc1_w.shape), full((1, fc1_w.shape[1])),
                         full(fc2_w.shape), full((1, fc2_w.shape[1])),
                         full(fc3_w.shape), full((1, fc3_w.shape[1]))]
            out_shape = [jax.ShapeDtypeStruct((B, N_HEADS, St, Ss), f32),
                         jax.ShapeDtypeStruct((B, 1, D), f32),
                         jax.ShapeDtypeStruct((B, 1, 2), f32)]
            out_specs = [pl.BlockSpec((G, N_HEADS, St, Ss),
                                      lambda b: (b, 0, 0, 0)),
                         pl.BlockSpec((G, 1, D), lambda b: (b, 0, 0)),
                         pl.BlockSpec((G, 1, 2), lambda b: (b, 0, 0))]
        else:
            out_shape = jax.ShapeDtypeStruct((B, St, D), f32)
            out_specs = pl.BlockSpec((G, St, D), lambda b: (b, 0, 0))

        res = pl.pallas_call(
            _make_layer_body(first, last, G),
            grid=(B // G,),
            in_specs=in_specs,
            out_specs=out_specs,
            out_shape=out_shape,
            compiler_params=pltpu.CompilerParams(
                dimension_semantics=("parallel",),
                vmem_limit_bytes=56 * 1024 * 1024,
            ),
        )(*ins)
        if last:
            attn, pooled3, label3 = res
        else:
            x = res

    return pooled3.reshape(B, D), attn, label3.reshape(B, 2)
